# baseline (speedup 1.0000x reference)
import jax
import jax.numpy as jnp
from jax import lax
from jax.experimental import pallas as pl
from jax.experimental.pallas import tpu as pltpu
from jax.experimental.pallas import tpu_sc as plsc

BB, SS, DD, FF, EE = 2, 2048, 1024, 1024, 64
TT = BB * SS
CAPACITY = 64
NSLOT = EE * CAPACITY
TBLK = 512
NTB = TT // TBLK
SCH = 512
NSC = NSLOT // SCH
FBLK = 256
NFB = FF // FBLK
NWRK = 32
GCH = 64


def _route_body(x_ref, wg_ref, dstc_ref, inv_ref, ssc_ref,
                counts_ref, inv_acc, occ_acc, ss_acc):
    i = pl.program_id(0)

    @pl.when(i == 0)
    def _init():
        counts_ref[...] = jnp.zeros_like(counts_ref)
        inv_acc[...] = jnp.zeros_like(inv_acc)
        occ_acc[...] = jnp.zeros_like(occ_acc)
        ss_acc[...] = jnp.zeros_like(ss_acc)

    logits = jnp.dot(x_ref[...], wg_ref[...],
                     preferred_element_type=jnp.float32)
    m = jnp.max(logits, axis=1, keepdims=True)
    sden = jnp.sum(jnp.exp(logits - m), axis=1, keepdims=True)
    score = 1.0 / sden
    e_iota = lax.broadcasted_iota(jnp.int32, (TBLK, EE), 1)
    cand = jnp.where(logits == m, e_iota, EE)
    idxc = jnp.min(cand, axis=1, keepdims=True)
    oh = (idxc == e_iota).astype(jnp.float32)
    r_io = lax.broadcasted_iota(jnp.int32, (TBLK, TBLK), 0)
    c_io = lax.broadcasted_iota(jnp.int32, (TBLK, TBLK), 1)
    tril = (r_io > c_io).astype(jnp.float32)
    within = jnp.dot(tril, oh, preferred_element_type=jnp.float32)
    cnt = counts_ref[...]
    posf = jnp.sum((within + cnt) * oh, axis=1, keepdims=True)
    counts_ref[...] = cnt + jnp.sum(oh, axis=0, keepdims=True)
    keep = posf < float(CAPACITY)
    dst = idxc * CAPACITY + posf.astype(jnp.int32)
    dstc = jnp.where(keep, dst, NSLOT)
    dstc_ref[...] = dstc
    seff = jnp.where(keep, score, 0.0)
    tids = (lax.broadcasted_iota(jnp.int32, (TBLK, 1), 0)
            + i * TBLK).astype(jnp.float32)
    inv_rows, occ_rows, ss_rows = [], [], []
    for s2 in range(NSC):
        slot_io = lax.broadcasted_iota(jnp.int32, (1, SCH), 1) + s2 * SCH
        msk = (dstc == slot_io).astype(jnp.float32)
        inv_rows.append(jnp.sum(msk * tids, axis=0, keepdims=True))
        occ_rows.append(jnp.sum(msk, axis=0, keepdims=True))
        ss_rows.append(jnp.sum(msk * seff, axis=0, keepdims=True))
    inv_acc[...] += jnp.concatenate(inv_rows, axis=0)
    occ_acc[...] += jnp.concatenate(occ_rows, axis=0)
    ss_acc[...] += jnp.concatenate(ss_rows, axis=0)
    inv_ref[...] = jnp.where(occ_acc[...] > 0, inv_acc[...], 0.0
                             ).astype(jnp.int32)
    ssc_ref[...] = ss_acc[...]


def _route(xf, wg):
    return pl.pallas_call(
        _route_body,
        grid=(NTB,),
        in_specs=[
            pl.BlockSpec((TBLK, DD), lambda i: (i, 0)),
            pl.BlockSpec((DD, EE), lambda i: (0, 0)),
        ],
        out_specs=[
            pl.BlockSpec((TBLK, 1), lambda i: (i, 0)),
            pl.BlockSpec((NSC, SCH), lambda i: (0, 0)),
            pl.BlockSpec((NSC, SCH), lambda i: (0, 0)),
        ],
        out_shape=[
            jax.ShapeDtypeStruct((TT, 1), jnp.int32),
            jax.ShapeDtypeStruct((NSC, SCH), jnp.int32),
            jax.ShapeDtypeStruct((NSC, SCH), jnp.float32),
        ],
        scratch_shapes=[
            pltpu.VMEM((1, EE), jnp.float32),
            pltpu.VMEM((NSC, SCH), jnp.float32),
            pltpu.VMEM((NSC, SCH), jnp.float32),
            pltpu.VMEM((NSC, SCH), jnp.float32),
        ],
    )(xf, wg)


def _ffn_body(disp_ref, w1_ref, b1_ref, w2_ref, b2_ref, ss_ref, o_ref,
              acc_ref):
    e = pl.program_id(0)
    f = pl.program_id(1)
    h = jnp.dot(disp_ref[...], w1_ref[0],
                preferred_element_type=jnp.float32) + b1_ref[0]
    h = jnp.maximum(h, 0.0)
    part = jnp.dot(h, w2_ref[0], preferred_element_type=jnp.float32)

    @pl.when(f == 0)
    def _first():
        acc_ref[...] = part

    @pl.when(f > 0)
    def _rest():
        acc_ref[...] += part

    @pl.when(f == NFB - 1)
    def _emit():
        scale = jnp.where(e < EE, ss_ref[...], 0.0)
        o_ref[...] = (acc_ref[...] + b2_ref[0]) * scale


def _ffn(disp, w1, b1, w2, b2, ssc):
    ec = lambda e: jnp.minimum(e, EE - 1)
    return pl.pallas_call(
        _ffn_body,
        grid=(EE + 1, NFB),
        in_specs=[
            pl.BlockSpec((CAPACITY, DD), lambda e, f: (ec(e), 0)),
            pl.BlockSpec((1, DD, FBLK), lambda e, f: (ec(e), 0, f)),
            pl.BlockSpec((1, 1, FBLK), lambda e, f: (ec(e), 0, f)),
            pl.BlockSpec((1, FBLK, DD), lambda e, f: (ec(e), f, 0)),
            pl.BlockSpec((1, 1, DD), lambda e, f: (ec(e), 0, 0)),
            pl.BlockSpec((CAPACITY, 1), lambda e, f: (ec(e), 0)),
        ],
        out_specs=pl.BlockSpec((CAPACITY, DD), lambda e, f: (e, 0)),
        out_shape=jax.ShapeDtypeStruct((NSLOT + CAPACITY, DD), jnp.float32),
        scratch_shapes=[pltpu.VMEM((CAPACITY, DD), jnp.float32)],
    )(disp, w1, b1.reshape(EE, 1, FF), w2, b2.reshape(EE, 1, DD), ssc)


def _gather_body(idx_hbm, src_hbm, out_hbm, idx_v, rows_v, sem):
    wid = lax.axis_index("s") * 2 + lax.axis_index("c")
    for cch in range(TT // NWRK // GCH):
        base = wid * (TT // NWRK) + cch * GCH
        pltpu.sync_copy(idx_hbm.at[pl.ds(base, GCH)], idx_v)
        pltpu.async_copy(src_hbm.at[idx_v], rows_v, sem).wait()
        pltpu.sync_copy(rows_v, out_hbm.at[pl.ds(base, GCH)])


def _row_gather(idx, src):
    mesh = plsc.VectorSubcoreMesh(core_axis_name="c", subcore_axis_name="s")
    return pl.kernel(
        _gather_body,
        out_type=jax.ShapeDtypeStruct((TT, DD), jnp.float32),
        mesh=mesh,
        scratch_types=[
            pltpu.VMEM((GCH,), jnp.int32),
            pltpu.VMEM((GCH, DD), jnp.float32),
            pltpu.SemaphoreType.DMA,
        ],
    )(idx, src)


def kernel(x, Wg, W1, b1, W2, b2):
    xf = x.reshape(TT, DD)
    dstc, inv, ssc = _route(xf, Wg)
    disp = _row_gather(inv.reshape(TT), xf)
    o = _ffn(disp, W1, b1, W2, b2, ssc.reshape(NSLOT, 1))
    y = _row_gather(dstc.reshape(TT), o)
    return y.reshape(BB, SS, DD)
